# baseline (speedup 1.0000x reference)
import functools

import jax
import jax.numpy as jnp
from jax import lax
from jax.experimental import pallas as pl
from jax.experimental.pallas import tpu as pltpu
from jax.experimental.pallas import tpu_sc as plsc

E = 8
TOPK = 2
D = 2048
F = 1408
T = 8192

BT = 1024
BD = 256
NDC = D // BD
NP = T * TOPK + E * BT
NB = NP // BT


def _up_body(be_ref, xs_ref, w1_ref, w3_ref, h_ref, g_ref, u_ref):
    dstep = pl.program_id(1)
    xb = xs_ref[...].astype(jnp.bfloat16)
    pg = jnp.dot(xb, w1_ref[0].astype(jnp.bfloat16),
                 preferred_element_type=jnp.float32)
    pu = jnp.dot(xb, w3_ref[0].astype(jnp.bfloat16),
                 preferred_element_type=jnp.float32)

    @pl.when(dstep == 0)
    def _init():
        g_ref[...] = pg
        u_ref[...] = pu

    @pl.when(dstep != 0)
    def _acc():
        g_ref[...] += pg
        u_ref[...] += pu

    @pl.when(dstep == NDC - 1)
    def _act():
        g = g_ref[...]
        h_ref[...] = ((g * jax.nn.sigmoid(g)) * u_ref[...]).astype(jnp.bfloat16)


_grouped_up = pl.pallas_call(
    _up_body,
    grid_spec=pltpu.PrefetchScalarGridSpec(
        num_scalar_prefetch=1,
        grid=(NB, NDC),
        in_specs=[
            pl.BlockSpec((BT, BD), lambda b, d, be: (b, d)),
            pl.BlockSpec((1, BD, F), lambda b, d, be: (be[b], d, 0)),
            pl.BlockSpec((1, BD, F), lambda b, d, be: (be[b], d, 0)),
        ],
        out_specs=pl.BlockSpec((BT, F), lambda b, d, be: (b, 0)),
        scratch_shapes=[pltpu.VMEM((BT, F), jnp.float32),
                        pltpu.VMEM((BT, F), jnp.float32)],
    ),
    out_shape=jax.ShapeDtypeStruct((NP, F), jnp.bfloat16),
    compiler_params=pltpu.CompilerParams(
        dimension_semantics=("arbitrary", "arbitrary")),
)


def _down_body(be_ref, h_ref, w2_ref, ys_ref):
    ys_ref[...] = jnp.dot(h_ref[...], w2_ref[0].astype(jnp.bfloat16),
                          preferred_element_type=jnp.float32)


_grouped_down = pl.pallas_call(
    _down_body,
    grid_spec=pltpu.PrefetchScalarGridSpec(
        num_scalar_prefetch=1,
        grid=(NB,),
        in_specs=[
            pl.BlockSpec((BT, F), lambda b, be: (b, 0)),
            pl.BlockSpec((1, F, D), lambda b, be: (be[b], 0, 0)),
        ],
        out_specs=pl.BlockSpec((BT, D), lambda b, be: (b, 0)),
    ),
    out_shape=jax.ShapeDtypeStruct((NP, D), jnp.float32),
    compiler_params=pltpu.CompilerParams(
        dimension_semantics=("arbitrary",)),
)

_NW = 32
_TPW = T // _NW

_ACH = 16
_NCH = _TPW // _ACH

_SC_MESH = plsc.VectorSubcoreMesh(core_axis_name="c", subcore_axis_name="s")


@functools.partial(
    pl.kernel,
    mesh=_SC_MESH,
    out_type=jax.ShapeDtypeStruct((NP, D), jnp.float32),
    scratch_types=[
        pltpu.VMEM((_NCH, _ACH), jnp.int32),
        pltpu.VMEM((_NCH, _ACH), jnp.int32),
        pltpu.VMEM((_ACH, D), jnp.float32),
        pltpu.SemaphoreType.DMA,
    ],
)
def _sc_scatter_x(x_hbm, pos0_hbm, pos1_hbm, xs_hbm, p0_v, p1_v, xbuf, sem):
    wid = lax.axis_index("s") * 2 + lax.axis_index("c")
    tok0 = wid * _TPW
    pltpu.sync_copy(pos0_hbm.at[wid], p0_v)
    pltpu.sync_copy(pos1_hbm.at[wid], p1_v)

    def chunk(c, carry):
        pltpu.sync_copy(x_hbm.at[pl.ds(tok0 + c * _ACH, _ACH)], xbuf)
        cp0 = pltpu.async_copy(xbuf, xs_hbm.at[p0_v.at[c]], sem)
        cp1 = pltpu.async_copy(xbuf, xs_hbm.at[p1_v.at[c]], sem)
        cp0.wait()
        cp1.wait()
        return carry

    lax.fori_loop(0, _NCH, chunk, 0)


_BCH = 16
_NBC = _TPW // _BCH


@functools.partial(
    pl.kernel,
    mesh=_SC_MESH,
    out_type=jax.ShapeDtypeStruct((T, D), jnp.float32),
    scratch_types=[
        pltpu.VMEM((_NBC, _BCH), jnp.int32),
        pltpu.VMEM((_NBC, _BCH), jnp.int32),
        pltpu.VMEM((_BCH, 16), jnp.float32),
        pltpu.VMEM((_BCH, 16), jnp.float32),
        pltpu.VMEM((_BCH, D), jnp.float32),
        pltpu.VMEM((_BCH, D), jnp.float32),
        pltpu.SemaphoreType.DMA,
    ],
)
def _sc_combine(ys_hbm, pos0_hbm, pos1_hbm, tw0_hbm, tw1_hbm, out_hbm,
                p0_v, p1_v, w0_v, w1_v, abuf, bbuf, sem):
    wid = lax.axis_index("s") * 2 + lax.axis_index("c")
    tok0 = wid * _TPW
    pltpu.sync_copy(pos0_hbm.at[wid], p0_v)
    pltpu.sync_copy(pos1_hbm.at[wid], p1_v)

    def chunk(c, carry):
        cpa = pltpu.async_copy(ys_hbm.at[p0_v.at[c]], abuf, sem)
        cpb = pltpu.async_copy(ys_hbm.at[p1_v.at[c]], bbuf, sem)
        pltpu.sync_copy(tw0_hbm.at[pl.ds(tok0 + c * _BCH, _BCH)], w0_v)
        pltpu.sync_copy(tw1_hbm.at[pl.ds(tok0 + c * _BCH, _BCH)], w1_v)
        cpa.wait()
        cpb.wait()

        def row(r, carry2):
            w0 = w0_v[r]
            w1 = w1_v[r]

            def col(j, carry3):
                sl = pl.ds(j * 16, 16)
                abuf[r, sl] = w0 * abuf[r, sl] + w1 * bbuf[r, sl]
                return carry3

            return lax.fori_loop(0, D // 16, col, carry2)

        lax.fori_loop(0, _BCH, row, carry)
        pltpu.sync_copy(abuf, out_hbm.at[pl.ds(tok0 + c * _BCH, _BCH)])
        return carry

    lax.fori_loop(0, _NBC, chunk, 0)


def kernel(x, gate_w, w1, w3, w2):
    logits = x @ gate_w
    probs = jax.nn.softmax(logits, axis=-1)
    tw, ti = jax.lax.top_k(probs, TOPK)
    tw = tw / jnp.sum(tw, axis=-1, keepdims=True)
    e0, e1 = ti[:, 0], ti[:, 1]

    oh = (jax.nn.one_hot(e0, E, dtype=jnp.int32)
          + jax.nn.one_hot(e1, E, dtype=jnp.int32))
    cinc = jnp.cumsum(oh, axis=0)
    cexc = cinc - oh
    total = cinc[-1]
    padded = ((total + BT - 1) // BT) * BT
    ends = jnp.cumsum(padded)
    base = ends - padded
    tarange = jnp.arange(T)
    pos0 = base[e0] + cexc[tarange, e0]
    pos1 = base[e1] + cexc[tarange, e1]

    block_expert = jnp.minimum(
        jnp.searchsorted(ends, jnp.arange(NB, dtype=jnp.int32) * BT,
                         side="right"),
        E - 1).astype(jnp.int32)

    pos0 = pos0.astype(jnp.int32)
    pos1 = pos1.astype(jnp.int32)
    xs = _sc_scatter_x(x,
                       pos0.reshape(_NW, _NCH, _ACH),
                       pos1.reshape(_NW, _NCH, _ACH))
    h = _grouped_up(block_expert, xs, w1, w3)
    ys = _grouped_down(block_expert, h, w2)
    tw0r = jnp.broadcast_to(tw[:, 0:1], (T, 16))
    tw1r = jnp.broadcast_to(tw[:, 1:2], (T, 16))
    out = _sc_combine(ys,
                      pos0.reshape(_NW, _NBC, _BCH),
                      pos1.reshape(_NW, _NBC, _BCH),
                      tw0r, tw1r)
    return out


# speedup vs baseline: 1.2121x; 1.2121x over previous
import functools

import jax
import jax.numpy as jnp
from jax import lax
from jax.experimental import pallas as pl
from jax.experimental.pallas import tpu as pltpu
from jax.experimental.pallas import tpu_sc as plsc

E = 8
TOPK = 2
D = 2048
F = 1408
T = 8192

BT = 1024
BD = 1024
NDC = D // BD
NP = T * TOPK + E * BT
NB = NP // BT


def _up_body(be_ref, xs_ref, w1_ref, w3_ref, h_ref, g_ref, u_ref):
    dstep = pl.program_id(1)
    xb = xs_ref[...].astype(jnp.bfloat16)
    pg = jnp.dot(xb, w1_ref[0].astype(jnp.bfloat16),
                 preferred_element_type=jnp.float32)
    pu = jnp.dot(xb, w3_ref[0].astype(jnp.bfloat16),
                 preferred_element_type=jnp.float32)

    @pl.when(dstep == 0)
    def _init():
        g_ref[...] = pg
        u_ref[...] = pu

    @pl.when(dstep != 0)
    def _acc():
        g_ref[...] += pg
        u_ref[...] += pu

    @pl.when(dstep == NDC - 1)
    def _act():
        g = g_ref[...]
        h_ref[...] = ((g * jax.nn.sigmoid(g)) * u_ref[...]).astype(jnp.bfloat16)


_grouped_up = pl.pallas_call(
    _up_body,
    grid_spec=pltpu.PrefetchScalarGridSpec(
        num_scalar_prefetch=1,
        grid=(NB, NDC),
        in_specs=[
            pl.BlockSpec((BT, BD), lambda b, d, be: (b, d)),
            pl.BlockSpec((1, BD, F), lambda b, d, be: (be[b], d, 0)),
            pl.BlockSpec((1, BD, F), lambda b, d, be: (be[b], d, 0)),
        ],
        out_specs=pl.BlockSpec((BT, F), lambda b, d, be: (b, 0)),
        scratch_shapes=[pltpu.VMEM((BT, F), jnp.float32),
                        pltpu.VMEM((BT, F), jnp.float32)],
    ),
    out_shape=jax.ShapeDtypeStruct((NP, F), jnp.bfloat16),
    compiler_params=pltpu.CompilerParams(
        dimension_semantics=("arbitrary", "arbitrary")),
)


def _down_body(be_ref, h_ref, w2_ref, ys_ref):
    ys_ref[...] = jnp.dot(h_ref[...], w2_ref[0].astype(jnp.bfloat16),
                          preferred_element_type=jnp.float32)


_grouped_down = pl.pallas_call(
    _down_body,
    grid_spec=pltpu.PrefetchScalarGridSpec(
        num_scalar_prefetch=1,
        grid=(NB,),
        in_specs=[
            pl.BlockSpec((BT, F), lambda b, be: (b, 0)),
            pl.BlockSpec((1, F, D), lambda b, be: (be[b], 0, 0)),
        ],
        out_specs=pl.BlockSpec((BT, D), lambda b, be: (b, 0)),
    ),
    out_shape=jax.ShapeDtypeStruct((NP, D), jnp.float32),
    compiler_params=pltpu.CompilerParams(
        dimension_semantics=("arbitrary",)),
)

_NW = 32
_TPW = T // _NW

_ACH = 16
_NCH = _TPW // _ACH

_SC_MESH = plsc.VectorSubcoreMesh(core_axis_name="c", subcore_axis_name="s")


@functools.partial(
    pl.kernel,
    mesh=_SC_MESH,
    out_type=jax.ShapeDtypeStruct((NP, D), jnp.float32),
    scratch_types=[
        pltpu.VMEM((_NCH, _ACH), jnp.int32),
        pltpu.VMEM((_NCH, _ACH), jnp.int32),
        pltpu.VMEM((_ACH, D), jnp.float32),
        pltpu.SemaphoreType.DMA,
    ],
)
def _sc_scatter_x(x_hbm, pos0_hbm, pos1_hbm, xs_hbm, p0_v, p1_v, xbuf, sem):
    wid = lax.axis_index("s") * 2 + lax.axis_index("c")
    tok0 = wid * _TPW
    pltpu.sync_copy(pos0_hbm.at[wid], p0_v)
    pltpu.sync_copy(pos1_hbm.at[wid], p1_v)

    def chunk(c, carry):
        pltpu.sync_copy(x_hbm.at[pl.ds(tok0 + c * _ACH, _ACH)], xbuf)
        cp0 = pltpu.async_copy(xbuf, xs_hbm.at[p0_v.at[c]], sem)
        cp1 = pltpu.async_copy(xbuf, xs_hbm.at[p1_v.at[c]], sem)
        cp0.wait()
        cp1.wait()
        return carry

    lax.fori_loop(0, _NCH, chunk, 0)


_BCH = 16
_NBC = _TPW // _BCH


@functools.partial(
    pl.kernel,
    mesh=_SC_MESH,
    out_type=jax.ShapeDtypeStruct((T, D), jnp.float32),
    scratch_types=[
        pltpu.VMEM((_NBC, _BCH), jnp.int32),
        pltpu.VMEM((_NBC, _BCH), jnp.int32),
        pltpu.VMEM((_BCH, 16), jnp.float32),
        pltpu.VMEM((_BCH, 16), jnp.float32),
        pltpu.VMEM((_BCH, D), jnp.float32),
        pltpu.VMEM((_BCH, D), jnp.float32),
        pltpu.SemaphoreType.DMA,
    ],
)
def _sc_combine(ys_hbm, pos0_hbm, pos1_hbm, tw0_hbm, tw1_hbm, out_hbm,
                p0_v, p1_v, w0_v, w1_v, abuf, bbuf, sem):
    wid = lax.axis_index("s") * 2 + lax.axis_index("c")
    tok0 = wid * _TPW
    pltpu.sync_copy(pos0_hbm.at[wid], p0_v)
    pltpu.sync_copy(pos1_hbm.at[wid], p1_v)

    def chunk(c, carry):
        cpa = pltpu.async_copy(ys_hbm.at[p0_v.at[c]], abuf, sem)
        cpb = pltpu.async_copy(ys_hbm.at[p1_v.at[c]], bbuf, sem)
        pltpu.sync_copy(tw0_hbm.at[pl.ds(tok0 + c * _BCH, _BCH)], w0_v)
        pltpu.sync_copy(tw1_hbm.at[pl.ds(tok0 + c * _BCH, _BCH)], w1_v)
        cpa.wait()
        cpb.wait()

        def row(r, carry2):
            w0 = w0_v[r]
            w1 = w1_v[r]

            def col(j, carry3):
                sl = pl.ds(j * 16, 16)
                abuf[r, sl] = w0 * abuf[r, sl] + w1 * bbuf[r, sl]
                return carry3

            return lax.fori_loop(0, D // 16, col, carry2)

        lax.fori_loop(0, _BCH, row, carry)
        pltpu.sync_copy(abuf, out_hbm.at[pl.ds(tok0 + c * _BCH, _BCH)])
        return carry

    lax.fori_loop(0, _NBC, chunk, 0)


def kernel(x, gate_w, w1, w3, w2):
    logits = x @ gate_w
    probs = jax.nn.softmax(logits, axis=-1)
    tw, ti = jax.lax.top_k(probs, TOPK)
    tw = tw / jnp.sum(tw, axis=-1, keepdims=True)
    e0, e1 = ti[:, 0], ti[:, 1]

    oh = (jax.nn.one_hot(e0, E, dtype=jnp.int32)
          + jax.nn.one_hot(e1, E, dtype=jnp.int32))
    cinc = jnp.cumsum(oh, axis=0)
    cexc = cinc - oh
    total = cinc[-1]
    padded = ((total + BT - 1) // BT) * BT
    ends = jnp.cumsum(padded)
    base = ends - padded
    tarange = jnp.arange(T)
    pos0 = base[e0] + cexc[tarange, e0]
    pos1 = base[e1] + cexc[tarange, e1]

    block_expert = jnp.minimum(
        jnp.searchsorted(ends, jnp.arange(NB, dtype=jnp.int32) * BT,
                         side="right"),
        E - 1).astype(jnp.int32)

    pos0 = pos0.astype(jnp.int32)
    pos1 = pos1.astype(jnp.int32)
    xs = _sc_scatter_x(x,
                       pos0.reshape(_NW, _NCH, _ACH),
                       pos1.reshape(_NW, _NCH, _ACH))
    h = _grouped_up(block_expert, xs, w1, w3)
    ys = _grouped_down(block_expert, h, w2)
    tw0r = jnp.broadcast_to(tw[:, 0:1], (T, 16))
    tw1r = jnp.broadcast_to(tw[:, 1:2], (T, 16))
    out = _sc_combine(ys,
                      pos0.reshape(_NW, _NBC, _BCH),
                      pos1.reshape(_NW, _NBC, _BCH),
                      tw0r, tw1r)
    return out
